# Initial kernel scaffold; baseline (speedup 1.0000x reference)
import jax
import jax.numpy as jnp
from jax.experimental import pallas as pl


def kernel(vision_embeddings, text_embeddings, vision_indices):
    raise NotImplementedError("write your pallas kernel here")


# speedup vs baseline: 1.4881x; 1.4881x over previous
import functools

import jax
import jax.numpy as jnp
from jax import lax
from jax.experimental import pallas as pl
from jax.experimental.pallas import tpu as pltpu
from jax.experimental.pallas import tpu_sc as plsc

_NC = 2
_NS = 16
_NW = _NC * _NS


def _prep_body(row_ref, col_ref, adj_ref, *, seq_len):
    b = pl.program_id(0)
    row = row_ref[0]
    col = col_ref[0]
    nv = col.shape[0]
    eq = col == row
    ii = lax.broadcasted_iota(jnp.int32, (nv, nv), 0)
    jj = lax.broadcasted_iota(jnp.int32, (nv, nv), 1)
    dup_later = jnp.any(eq & (jj > ii), axis=1, keepdims=True)
    keep = (col != 0) & jnp.logical_not(dup_later)
    adj_ref[0] = jnp.where(keep, col + b * seq_len, 0)


@functools.partial(jax.jit, static_argnames=("seq_len",))
def _prep(vision_indices, *, seq_len):
    b, nv = vision_indices.shape
    row3 = vision_indices.reshape(b, 1, nv)
    col3 = vision_indices.reshape(b, nv, 1)
    return pl.pallas_call(
        functools.partial(_prep_body, seq_len=seq_len),
        grid=(b,),
        in_specs=[
            pl.BlockSpec((1, 1, nv), lambda i: (i, 0, 0)),
            pl.BlockSpec((1, nv, 1), lambda i: (i, 0, 0)),
        ],
        out_specs=pl.BlockSpec((1, nv, 1), lambda i: (i, 0, 0)),
        out_shape=jax.ShapeDtypeStruct((b, nv, 1), jnp.int32),
    )(row3, col3)


def _make_sc_kernels(tok, hdim, cw):
    tpw = tok // _NW
    nch = tpw // cw
    mesh = plsc.VectorSubcoreMesh(
        core_axis_name="c",
        subcore_axis_name="s",
        num_cores=_NC,
        num_subcores=_NS,
    )

    @functools.partial(
        pl.kernel,
        mesh=mesh,
        out_type=(),
        scratch_types=[
            pltpu.VMEM((cw,), jnp.int32),
            pltpu.VMEM((cw, hdim), jnp.float32),
            pltpu.SemaphoreType.DMA,
        ],
    )
    def scatter_k(vision_hbm, adj_hbm, out_ref, idx_v, rows_v, sem):
        wid = lax.axis_index("s") * _NC + lax.axis_index("c")
        for ch in range(nch):
            base = wid * tpw + ch * cw
            pltpu.sync_copy(adj_hbm.at[wid * nch + ch], idx_v)
            pltpu.sync_copy(vision_hbm.at[pl.ds(base, cw)], rows_v)
            pltpu.async_copy(rows_v, out_ref.at[idx_v], sem).wait()

    @functools.partial(
        pl.kernel,
        mesh=mesh,
        out_type=(),
        scratch_types=[pltpu.VMEM((1, hdim), jnp.float32)],
    )
    def restore_k(text_hbm, out_ref, row_v):
        c = lax.axis_index("c")
        s = lax.axis_index("s")

        @pl.when((c == 0) & (s == 0))
        def _():
            pltpu.sync_copy(text_hbm.at[pl.ds(0, 1)], row_v)
            pltpu.sync_copy(row_v, out_ref.at[pl.ds(0, 1)])

    return scatter_k, restore_k


def kernel(vision_embeddings, text_embeddings, vision_indices):
    b, s, h = text_embeddings.shape
    tok = vision_embeddings.shape[0]
    cw = 32
    assert tok % _NW == 0 and (tok // _NW) % cw == 0

    flat_text = text_embeddings.reshape(b * s, h)
    adj = _prep(vision_indices.astype(jnp.int32), seq_len=s)
    adj2 = adj.reshape(tok // cw, cw)

    scatter_k, restore_k = _make_sc_kernels(tok, h, cw)
    out_ref = jax.new_ref(flat_text)
    scatter_k(vision_embeddings, adj2, out_ref)
    restore_k(flat_text, out_ref)
    return out_ref[...].reshape(b, s, h)


# speedup vs baseline: 1.5096x; 1.0144x over previous
import functools

import jax
import jax.numpy as jnp
from jax import lax
from jax.experimental import pallas as pl
from jax.experimental.pallas import tpu as pltpu
from jax.experimental.pallas import tpu_sc as plsc

_NC = 2
_NS = 16
_NW = _NC * _NS


def _prep_body(row_ref, col_ref, adj_ref, *, seq_len):
    b = pl.program_id(0)
    row = row_ref[0]
    col = col_ref[0]
    nv = col.shape[0]
    eq = col == row
    ii = lax.broadcasted_iota(jnp.int32, (nv, nv), 0)
    jj = lax.broadcasted_iota(jnp.int32, (nv, nv), 1)
    dup_later = jnp.any(eq & (jj > ii), axis=1, keepdims=True)
    keep = (col != 0) & jnp.logical_not(dup_later)
    adj_ref[0] = jnp.where(keep, col + b * seq_len, 0)


@functools.partial(jax.jit, static_argnames=("seq_len",))
def _prep(vision_indices, *, seq_len):
    b, nv = vision_indices.shape
    row3 = vision_indices.reshape(b, 1, nv)
    col3 = vision_indices.reshape(b, nv, 1)
    return pl.pallas_call(
        functools.partial(_prep_body, seq_len=seq_len),
        grid=(b,),
        in_specs=[
            pl.BlockSpec((1, 1, nv), lambda i: (i, 0, 0)),
            pl.BlockSpec((1, nv, 1), lambda i: (i, 0, 0)),
        ],
        out_specs=pl.BlockSpec((1, nv, 1), lambda i: (i, 0, 0)),
        out_shape=jax.ShapeDtypeStruct((b, nv, 1), jnp.int32),
    )(row3, col3)


def _make_sc_kernels(tok, hdim, cw):
    tpw = tok // _NW
    nch = tpw // cw
    mesh = plsc.VectorSubcoreMesh(
        core_axis_name="c",
        subcore_axis_name="s",
        num_cores=_NC,
        num_subcores=_NS,
    )

    nbuf = 3

    @functools.partial(
        pl.kernel,
        mesh=mesh,
        out_type=(),
        scratch_types=[
            [pltpu.VMEM((cw,), jnp.int32) for _ in range(nbuf)],
            pltpu.VMEM((nbuf, cw, hdim), jnp.float32),
            [pltpu.SemaphoreType.DMA for _ in range(nbuf)],
            [pltpu.SemaphoreType.DMA for _ in range(nbuf)],
            [pltpu.SemaphoreType.DMA for _ in range(nbuf)],
        ],
    )
    def scatter_k(vision_hbm, adj_hbm, out_ref, idx_bufs, rows_v, sems_i,
                  sems_l, sems_s):
        wid = lax.axis_index("s") * _NC + lax.axis_index("c")
        idxd = [None] * nch
        loadd = [None] * nch
        scatd = [None] * nch

        def start_stage(ch):
            b = ch % nbuf
            if ch >= nbuf:
                scatd[ch - nbuf].wait()
            idxd[ch] = pltpu.async_copy(
                adj_hbm.at[wid * nch + ch], idx_bufs[b], sems_i[b]
            )
            loadd[ch] = pltpu.async_copy(
                vision_hbm.at[pl.ds(wid * tpw + ch * cw, cw)],
                rows_v.at[b],
                sems_l[b],
            )

        def issue_scatter(ch):
            b = ch % nbuf
            idxd[ch].wait()
            loadd[ch].wait()
            scatd[ch] = pltpu.async_copy(
                rows_v.at[b], out_ref.at[idx_bufs[b]], sems_s[b]
            )

        for ch in range(nch):
            start_stage(ch)
            if ch >= nbuf - 1:
                issue_scatter(ch - (nbuf - 1))
        for ch in range(max(0, nch - (nbuf - 1)), nch):
            issue_scatter(ch)
        for ch in range(max(0, nch - nbuf), nch):
            scatd[ch].wait()

    @functools.partial(
        pl.kernel,
        mesh=mesh,
        out_type=(),
        scratch_types=[pltpu.VMEM((1, hdim), jnp.float32)],
    )
    def restore_k(text_hbm, out_ref, row_v):
        c = lax.axis_index("c")
        s = lax.axis_index("s")

        @pl.when((c == 0) & (s == 0))
        def _():
            pltpu.sync_copy(text_hbm.at[pl.ds(0, 1)], row_v)
            pltpu.sync_copy(row_v, out_ref.at[pl.ds(0, 1)])

    return scatter_k, restore_k


def kernel(vision_embeddings, text_embeddings, vision_indices):
    b, s, h = text_embeddings.shape
    tok = vision_embeddings.shape[0]
    cw = 16
    assert tok % _NW == 0 and (tok // _NW) % cw == 0

    flat_text = text_embeddings.reshape(b * s, h)
    adj = _prep(vision_indices.astype(jnp.int32), seq_len=s)
    adj2 = adj.reshape(tok // cw, cw)

    scatter_k, restore_k = _make_sc_kernels(tok, h, cw)
    out_ref = jax.new_ref(flat_text)
    scatter_k(vision_embeddings, adj2, out_ref)
    restore_k(flat_text, out_ref)
    return out_ref[...].reshape(b, s, h)


# speedup vs baseline: 1.6605x; 1.1000x over previous
import functools

import jax
import jax.numpy as jnp
from jax import lax
from jax.experimental import pallas as pl
from jax.experimental.pallas import tpu as pltpu
from jax.experimental.pallas import tpu_sc as plsc

_NC = 2
_NS = 16
_NW = _NC * _NS


def _prep_body(row_ref, col_ref, adj_ref, *, seq_len, tpw, rows_per_batch):
    b = pl.program_id(0)
    row = row_ref[0]
    col = col_ref[0]
    nv = row.shape[1]
    eq = col == row
    ii = lax.broadcasted_iota(jnp.int32, (nv, nv), 0)
    jj = lax.broadcasted_iota(jnp.int32, (nv, nv), 1)
    dup_later = jnp.any(eq & (ii > jj), axis=0, keepdims=True)
    keep = (row != 0) & jnp.logical_not(dup_later)
    jl = lax.broadcasted_iota(jnp.int32, (1, nv), 1)
    trash = (((b * nv + jl) // tpw) % _NC) * seq_len
    adj = jnp.where(keep, row + b * seq_len, trash)
    for k in range(rows_per_batch):
        w = nv // rows_per_batch
        adj_ref[k : k + 1, :] = adj[:, k * w : (k + 1) * w]


@functools.partial(jax.jit, static_argnames=("seq_len", "tpw"))
def _prep(vision_indices, *, seq_len, tpw):
    b, nv = vision_indices.shape
    rpb = nv // tpw
    row3 = vision_indices.reshape(b, 1, nv)
    col3 = vision_indices.reshape(b, nv, 1)
    return pl.pallas_call(
        functools.partial(
            _prep_body, seq_len=seq_len, tpw=tpw, rows_per_batch=rpb
        ),
        grid=(b,),
        in_specs=[
            pl.BlockSpec((1, 1, nv), lambda i: (i, 0, 0)),
            pl.BlockSpec((1, nv, 1), lambda i: (i, 0, 0)),
        ],
        out_specs=pl.BlockSpec((rpb, tpw), lambda i: (i, 0)),
        out_shape=jax.ShapeDtypeStruct((b * rpb, tpw), jnp.int32),
    )(row3, col3)


def _make_scatter(tok, hdim, cw, nbuf, trash_stride):
    tpw = tok // _NW
    nch = tpw // cw
    mesh = plsc.VectorSubcoreMesh(
        core_axis_name="c",
        subcore_axis_name="s",
        num_cores=_NC,
        num_subcores=_NS,
    )

    @functools.partial(
        pl.kernel,
        mesh=mesh,
        out_type=(),
        scratch_types=[
            pltpu.VMEM((nch, cw), jnp.int32),
            pltpu.VMEM((nbuf, cw, hdim), jnp.float32),
            pltpu.VMEM((1, hdim), jnp.float32),
            [pltpu.SemaphoreType.DMA for _ in range(nbuf)],
            [pltpu.SemaphoreType.DMA for _ in range(nbuf)],
        ],
    )
    def scatter_k(vision_hbm, adj_hbm, text_hbm, out_ref, idx_v, rows_v,
                  row_v, sems_l, sems_s):
        c = lax.axis_index("c")
        s = lax.axis_index("s")
        wid = s * _NC + c
        pltpu.sync_copy(adj_hbm.at[wid], idx_v)
        loadd = [None] * nch
        scatd = [None] * nch

        def start_load(ch):
            b = ch % nbuf
            if ch >= nbuf:
                scatd[ch - nbuf].wait()
            loadd[ch] = pltpu.async_copy(
                vision_hbm.at[pl.ds(wid * tpw + ch * cw, cw)],
                rows_v.at[b],
                sems_l[b],
            )

        def issue_scatter(ch):
            b = ch % nbuf
            loadd[ch].wait()
            scatd[ch] = pltpu.async_copy(
                rows_v.at[b], out_ref.at[idx_v.at[ch]], sems_s[b]
            )

        for ch in range(nch):
            start_load(ch)
            if ch >= nbuf - 1:
                issue_scatter(ch - (nbuf - 1))
        for ch in range(max(0, nch - (nbuf - 1)), nch):
            issue_scatter(ch)
        for ch in range(max(0, nch - nbuf), nch):
            scatd[ch].wait()

        plsc.subcore_barrier()

        @pl.when(s == 0)
        def _():
            trash0 = pl.multiple_of(c * trash_stride, 8)
            pltpu.sync_copy(text_hbm.at[pl.ds(trash0, 1)], row_v)
            pltpu.sync_copy(row_v, out_ref.at[pl.ds(trash0, 1)])

    return scatter_k


def kernel(vision_embeddings, text_embeddings, vision_indices):
    b, s, h = text_embeddings.shape
    tok = vision_embeddings.shape[0]
    tpw = tok // _NW
    cw = 8
    nbuf = 6
    assert tok % _NW == 0 and tpw % cw == 0

    flat_text = text_embeddings.reshape(b * s, h)
    adj = _prep(vision_indices.astype(jnp.int32), seq_len=s, tpw=tpw)
    adj3 = adj.reshape(_NW, tpw // cw, cw)

    scatter_k = _make_scatter(tok, h, cw, nbuf, s)
    out_ref = jax.new_ref(flat_text)
    scatter_k(vision_embeddings, adj3, flat_text, out_ref)
    return out_ref[...].reshape(b, s, h)


# speedup vs baseline: 1.7142x; 1.0323x over previous
import functools

import jax
import jax.numpy as jnp
from jax import lax
from jax.experimental import pallas as pl
from jax.experimental.pallas import tpu as pltpu
from jax.experimental.pallas import tpu_sc as plsc

_NC = 2
_NS = 16
_NW = _NC * _NS


def _prep_body(row_ref, adj_ref, *, seq_len, tpw, rows_per_batch):
    b = pl.program_id(0)
    row = row_ref[0]
    nv = row.shape[1]
    col = jnp.reshape(row, (nv, 1))
    eq = col == row
    ii = lax.broadcasted_iota(jnp.int32, (nv, nv), 0)
    jj = lax.broadcasted_iota(jnp.int32, (nv, nv), 1)
    dup_later = jnp.any(eq & (ii > jj), axis=0, keepdims=True)
    keep = (row != 0) & jnp.logical_not(dup_later)
    jl = lax.broadcasted_iota(jnp.int32, (1, nv), 1)
    trash = (((b * nv + jl) // tpw) % _NC) * seq_len
    adj = jnp.where(keep, row + b * seq_len, trash)
    for k in range(rows_per_batch):
        w = nv // rows_per_batch
        adj_ref[k : k + 1, :] = adj[:, k * w : (k + 1) * w]


@functools.partial(jax.jit, static_argnames=("seq_len", "tpw"))
def _prep(vision_indices, *, seq_len, tpw):
    b, nv = vision_indices.shape
    rpb = nv // tpw
    row3 = vision_indices.reshape(b, 1, nv)
    return pl.pallas_call(
        functools.partial(
            _prep_body, seq_len=seq_len, tpw=tpw, rows_per_batch=rpb
        ),
        grid=(b,),
        in_specs=[
            pl.BlockSpec((1, 1, nv), lambda i: (i, 0, 0)),
        ],
        out_specs=pl.BlockSpec((rpb, tpw), lambda i: (i, 0)),
        out_shape=jax.ShapeDtypeStruct((b * rpb, tpw), jnp.int32),
    )(row3)


def _make_scatter(tok, hdim, cw, nbuf, trash_stride):
    tpw = tok // _NW
    nch = tpw // cw
    mesh = plsc.VectorSubcoreMesh(
        core_axis_name="c",
        subcore_axis_name="s",
        num_cores=_NC,
        num_subcores=_NS,
    )

    @functools.partial(
        pl.kernel,
        mesh=mesh,
        out_type=(),
        scratch_types=[
            pltpu.VMEM((nch, cw), jnp.int32),
            pltpu.VMEM((nbuf, cw, hdim), jnp.float32),
            pltpu.VMEM((1, hdim), jnp.float32),
            [pltpu.SemaphoreType.DMA for _ in range(nbuf)],
            [pltpu.SemaphoreType.DMA for _ in range(nbuf)],
        ],
    )
    def scatter_k(vision_hbm, adj_hbm, text_hbm, out_ref, idx_v, rows_v,
                  row_v, sems_l, sems_s):
        c = lax.axis_index("c")
        s = lax.axis_index("s")
        wid = s * _NC + c
        pltpu.sync_copy(adj_hbm.at[wid], idx_v)
        loadd = [None] * nch
        scatd = [None] * nch

        def start_load(ch):
            b = ch % nbuf
            if ch >= nbuf:
                scatd[ch - nbuf].wait()
            loadd[ch] = pltpu.async_copy(
                vision_hbm.at[pl.ds(wid * tpw + ch * cw, cw)],
                rows_v.at[b],
                sems_l[b],
            )

        def issue_scatter(ch):
            b = ch % nbuf
            loadd[ch].wait()
            scatd[ch] = pltpu.async_copy(
                rows_v.at[b], out_ref.at[idx_v.at[ch]], sems_s[b]
            )

        for ch in range(nch):
            start_load(ch)
            if ch >= nbuf - 1:
                issue_scatter(ch - (nbuf - 1))
        for ch in range(max(0, nch - (nbuf - 1)), nch):
            issue_scatter(ch)
        for ch in range(max(0, nch - nbuf), nch):
            scatd[ch].wait()

        plsc.subcore_barrier()

        @pl.when(s == 0)
        def _():
            trash0 = pl.multiple_of(c * trash_stride, 8)
            pltpu.sync_copy(text_hbm.at[pl.ds(trash0, 1)], row_v)
            pltpu.sync_copy(row_v, out_ref.at[pl.ds(trash0, 1)])

    return scatter_k


def kernel(vision_embeddings, text_embeddings, vision_indices):
    b, s, h = text_embeddings.shape
    tok = vision_embeddings.shape[0]
    tpw = tok // _NW
    cw = 8
    nbuf = 7
    assert tok % _NW == 0 and tpw % cw == 0

    flat_text = text_embeddings.reshape(b * s, h)
    adj = _prep(vision_indices.astype(jnp.int32), seq_len=s, tpw=tpw)
    adj3 = adj.reshape(_NW, tpw // cw, cw)

    scatter_k = _make_scatter(tok, h, cw, nbuf, s)
    out_ref = jax.new_ref(flat_text)
    scatter_k(vision_embeddings, adj3, flat_text, out_ref)
    return out_ref[...].reshape(b, s, h)
